# baseline (speedup 1.0000x reference)
import jax
import jax.numpy as jnp
from jax import lax
from jax.experimental import pallas as pl
from jax.experimental.pallas import tpu as pltpu

_N = 20000
_PAD_N = 20480
_ROWS = 160
_LANES = 128
_CONF = 0.25
_IOU_T = 0.45
_MAX_DET = 300
_BIG = 2**30


def _fold(av, ai, bv, bi):
    t = (bv > av) | ((bv == av) & (bi < ai))
    return jnp.where(t, bv, av), jnp.where(t, bi, ai)


def _argmax_lanes(colmax, linc):
    v, i = colmax, linc
    for sh in (64, 32, 16, 8, 4, 2, 1):
        v2 = pltpu.roll(v, _LANES - sh, 1)
        i2 = pltpu.roll(i, _LANES - sh, 1)
        v, i = _fold(v, i, v2, i2)
    return v[:, :1], i[0, 0]


def _select(s, rowiota, laneiota):
    colmax = jnp.max(s, axis=0, keepdims=True)
    rowsel = jnp.min(
        jnp.where(s >= colmax, rowiota, _BIG), axis=0, keepdims=True
    )
    linc = rowsel * _LANES + laneiota
    return _argmax_lanes(colmax, linc)


def _nms_kernel(planes_ref, packed_ref, scores_ref, out_ref):
    cx = planes_ref[0]
    cy = planes_ref[1]
    w = planes_ref[2]
    h = planes_ref[3]
    x1 = cx - w / 2
    y1 = cy - h / 2
    x2 = cx + w / 2
    y2 = cy + h / 2
    area = jnp.clip(x2 - x1, 0.0) * jnp.clip(y2 - y1, 0.0)

    raw_s = scores_ref[...]
    s0 = jnp.where(raw_s > _CONF, raw_s, 0.0)

    lin = (
        lax.broadcasted_iota(jnp.int32, (_ROWS, _LANES), 0) * _LANES
        + lax.broadcasted_iota(jnp.int32, (_ROWS, _LANES), 1)
    )
    rowiota = lax.broadcasted_iota(jnp.int32, (_ROWS, _LANES), 0)
    laneiota = lax.broadcasted_iota(jnp.int32, (1, _LANES), 1)
    lane8 = lax.broadcasted_iota(jnp.int32, (1, 8), 1)

    m0, idx0 = _select(s0, rowiota, laneiota)

    def body(i, carry):
        s, m, idx = carry
        g = packed_ref[pl.ds(idx // 2, 1), :]
        hi = idx % 2 == 1
        bcx = jnp.where(hi, g[0, 4], g[0, 0])
        bcy = jnp.where(hi, g[0, 5], g[0, 1])
        bw = jnp.where(hi, g[0, 6], g[0, 2])
        bh = jnp.where(hi, g[0, 7], g[0, 3])
        bx1 = bcx - bw / 2
        by1 = bcy - bh / 2
        bx2 = bcx + bw / 2
        by2 = bcy + bh / 2
        ix1 = jnp.maximum(bx1, x1)
        iy1 = jnp.maximum(by1, y1)
        ix2 = jnp.minimum(bx2, x2)
        iy2 = jnp.minimum(by2, y2)
        inter = jnp.clip(ix2 - ix1, 0.0) * jnp.clip(iy2 - iy1, 0.0)
        area_a = jnp.clip(bx2 - bx1, 0.0) * jnp.clip(by2 - by1, 0.0)
        iou = inter / (area_a + area - inter + 1e-9)
        s = jnp.where((iou > _IOU_T) | (lin == idx), 0.0, s)
        m_next, idx_next = _select(s, rowiota, laneiota)
        vf = jnp.where(m > 0.0, 1.0, 0.0)
        row = (
            jnp.where(lane8 == 0, bx1, 0.0)
            + jnp.where(lane8 == 1, by1, 0.0)
            + jnp.where(lane8 == 2, bx2, 0.0)
            + jnp.where(lane8 == 3, by2, 0.0)
            + jnp.where(lane8 == 4, m, 0.0)
        ) * vf
        out_ref[pl.ds(i, 1), :] = row
        return (s, m_next, idx_next)

    lax.fori_loop(0, _MAX_DET, body, (s0, m0, idx0))


def kernel(boxes, scores):
    bp = jnp.pad(boxes, ((0, _PAD_N - _N), (0, 0)))
    planes = bp.T.reshape(4, _ROWS, _LANES)
    packed = bp.reshape(_PAD_N // 2, 8)
    s = jnp.pad(scores, (0, _PAD_N - _N)).reshape(_ROWS, _LANES)
    out = pl.pallas_call(
        _nms_kernel,
        out_shape=jax.ShapeDtypeStruct((_MAX_DET, 8), jnp.float32),
    )(planes, packed, s)
    return out[:, :5]


# speedup vs baseline: 1.4108x; 1.4108x over previous
import jax
import jax.numpy as jnp
from jax import lax
from jax.experimental import pallas as pl

_N = 20000
_PAD_N = 20480
_ROWS = 160
_LANES = 128
_CONF = 0.25
_IOU_T = 0.45
_MAX_DET = 300
_BIG = 2**30


def _nms_kernel(planes_ref, packed_ref, scores_ref, out_ref):
    cx = planes_ref[0]
    cy = planes_ref[1]
    w = planes_ref[2]
    h = planes_ref[3]
    x1 = cx - w / 2
    y1 = cy - h / 2
    x2 = cx + w / 2
    y2 = cy + h / 2
    area = jnp.clip(x2 - x1, 0.0) * jnp.clip(y2 - y1, 0.0)

    raw_s = scores_ref[...]
    s0 = jnp.where(raw_s > _CONF, raw_s, 0.0)

    lin = (
        lax.broadcasted_iota(jnp.int32, (_ROWS, _LANES), 0) * _LANES
        + lax.broadcasted_iota(jnp.int32, (_ROWS, _LANES), 1)
    )
    rowiota = lax.broadcasted_iota(jnp.int32, (_ROWS, _LANES), 0)
    laneiota = lax.broadcasted_iota(jnp.int32, (1, _LANES), 1)
    lane8 = lax.broadcasted_iota(jnp.int32, (1, 8), 1)

    def select(s):
        colmax = jnp.max(s, axis=0, keepdims=True)
        rowsel = jnp.min(
            jnp.where(s >= colmax, rowiota, _BIG), axis=0, keepdims=True
        )
        m = jnp.max(colmax)
        linc = rowsel * _LANES + laneiota
        idx = jnp.min(jnp.where(colmax >= m, linc, _BIG))
        return m, idx

    m0, idx0 = select(s0)

    def body(i, carry):
        s, m, idx = carry
        g = packed_ref[pl.ds(idx // 2, 1), :]
        hi = idx % 2 == 1
        bcx = jnp.where(hi, g[0, 4], g[0, 0])
        bcy = jnp.where(hi, g[0, 5], g[0, 1])
        bw = jnp.where(hi, g[0, 6], g[0, 2])
        bh = jnp.where(hi, g[0, 7], g[0, 3])
        bx1 = bcx - bw / 2
        by1 = bcy - bh / 2
        bx2 = bcx + bw / 2
        by2 = bcy + bh / 2
        ix1 = jnp.maximum(bx1, x1)
        iy1 = jnp.maximum(by1, y1)
        ix2 = jnp.minimum(bx2, x2)
        iy2 = jnp.minimum(by2, y2)
        inter = jnp.clip(ix2 - ix1, 0.0) * jnp.clip(iy2 - iy1, 0.0)
        area_a = jnp.clip(bx2 - bx1, 0.0) * jnp.clip(by2 - by1, 0.0)
        iou = inter / (area_a + area - inter + 1e-9)
        s = jnp.where((iou > _IOU_T) | (lin == idx), 0.0, s)
        m_next, idx_next = select(s)
        vf = jnp.where(m > 0.0, 1.0, 0.0)
        row = (
            jnp.where(lane8 == 0, bx1, 0.0)
            + jnp.where(lane8 == 1, by1, 0.0)
            + jnp.where(lane8 == 2, bx2, 0.0)
            + jnp.where(lane8 == 3, by2, 0.0)
            + jnp.where(lane8 == 4, m, 0.0)
        ) * vf
        out_ref[pl.ds(i, 1), :] = row
        return (s, m_next, idx_next)

    lax.fori_loop(0, _MAX_DET, body, (s0, m0, idx0))


def kernel(boxes, scores):
    bp = jnp.pad(boxes, ((0, _PAD_N - _N), (0, 0)))
    planes = bp.T.reshape(4, _ROWS, _LANES)
    packed = bp.reshape(_PAD_N // 2, 8)
    s = jnp.pad(scores, (0, _PAD_N - _N)).reshape(_ROWS, _LANES)
    out = pl.pallas_call(
        _nms_kernel,
        out_shape=jax.ShapeDtypeStruct((_MAX_DET, 8), jnp.float32),
    )(planes, packed, s)
    return out[:, :5]
